# Initial kernel scaffold; baseline (speedup 1.0000x reference)
import functools

import jax
import jax.numpy as jnp
from jax import lax
from jax.experimental import pallas as pl
from jax.experimental.pallas import tpu as pltpu
from jax.experimental.pallas import tpu_sc as plsc

_EPS = 1e-5
_B = 16384
_F = 26
_V = 100000
_D = 50
_NC = 13
_FD = _F * _D
_L1 = 512
_L2 = 256

_ROWS = _B * _F
_NUM_WORKERS = 32
_ROWS_PER_W = _ROWS // _NUM_WORKERS
_CHUNK = 832
_NCHUNKS = _ROWS_PER_W // _CHUNK


def _sc_gather(flat_tables, flat_idx):
    mesh = plsc.VectorSubcoreMesh(core_axis_name="c", subcore_axis_name="s")

    @functools.partial(
        pl.kernel,
        out_type=jax.ShapeDtypeStruct((_ROWS, _D), jnp.float32),
        mesh=mesh,
        scratch_types=[
            pltpu.VMEM((_ROWS_PER_W,), jnp.int32),
            pltpu.VMEM((_CHUNK, _D), jnp.float32),
            pltpu.SemaphoreType.DMA,
        ],
    )
    def gather_kernel(tab_hbm, idx_hbm, out_hbm, idx_v, rows_v, sem):
        wid = lax.axis_index("s") * 2 + lax.axis_index("c")
        base = wid * _ROWS_PER_W
        pltpu.sync_copy(idx_hbm.at[pl.ds(base, _ROWS_PER_W)], idx_v)

        @pl.loop(0, _NCHUNKS)
        def _(i):
            off = i * _CHUNK
            pltpu.async_copy(
                tab_hbm.at[idx_v.at[pl.ds(off, _CHUNK)]], rows_v, sem
            ).wait()
            pltpu.sync_copy(rows_v, out_hbm.at[pl.ds(base + off, _CHUNK)])

    return gather_kernel(flat_tables, flat_idx)


def _mlp_body(x_ref, xc_ref, gc_ref, bc_ref, w1a_ref, w1b_ref, b1_ref,
              g1_ref, bt1_ref, w2_ref, b2_ref, g2_ref, bt2_ref, wo_ref,
              bo_ref, o_ref):
    inv = (1.0 / jnp.sqrt(1.0 + _EPS)).astype(jnp.float32)
    xc = xc_ref[...] * (gc_ref[...] * inv) + bc_ref[...]
    h = jnp.dot(x_ref[...], w1a_ref[...], preferred_element_type=jnp.float32)
    h = h + jnp.dot(xc, w1b_ref[...], preferred_element_type=jnp.float32)
    h = jnp.maximum(h + b1_ref[...], 0.0)
    h = h * (g1_ref[...] * inv) + bt1_ref[...]
    h = jnp.maximum(
        jnp.dot(h, w2_ref[...], preferred_element_type=jnp.float32)
        + b2_ref[...], 0.0)
    h = h * (g2_ref[...] * inv) + bt2_ref[...]
    o_ref[...] = (
        jnp.dot(h, wo_ref[...], preferred_element_type=jnp.float32)
        + bo_ref[...])


def _tc_mlp(x, x_cont, g_cont, b_cont, W1, b1, g1, beta1, W2, b2, g2, beta2,
            Wo, bo):
    bt = 1024
    grid = (_B // bt,)
    row = lambda v: v.reshape(1, -1)
    full = lambda a: pl.BlockSpec(a.shape, lambda i: (0, 0))
    args = (
        x, x_cont, row(g_cont), row(b_cont),
        W1[:_FD], W1[_FD:], row(b1), row(g1), row(beta1),
        W2, row(b2), row(g2), row(beta2), Wo, row(bo),
    )
    in_specs = [
        pl.BlockSpec((bt, _FD), lambda i: (i, 0)),
        pl.BlockSpec((bt, _NC), lambda i: (i, 0)),
    ] + [full(a) for a in args[2:]]
    return pl.pallas_call(
        _mlp_body,
        grid=grid,
        in_specs=in_specs,
        out_specs=pl.BlockSpec((bt, 1), lambda i: (i, 0)),
        out_shape=jax.ShapeDtypeStruct((_B, 1), jnp.float32),
        compiler_params=pltpu.CompilerParams(
            dimension_semantics=("arbitrary",)),
    )(*args)


def kernel(x_cat, x_cont, tables, g_cont, b_cont, W1, b1, g1, beta1, W2, b2,
           g2, beta2, Wo, bo):
    flat_tables = tables.reshape(_F * _V, _D)
    offs = (jnp.arange(_F, dtype=jnp.int32) * _V)[None, :]
    flat_idx = (x_cat.astype(jnp.int32) + offs).reshape(_ROWS)
    emb = _sc_gather(flat_tables, flat_idx)
    x = emb.reshape(_B, _FD)
    return _tc_mlp(x, x_cont, g_cont, b_cont, W1, b1, g1, beta1, W2, b2, g2,
                   beta2, Wo, bo)


# speedup vs baseline: 4.2288x; 4.2288x over previous
import functools

import jax
import jax.numpy as jnp
from jax import lax
from jax.experimental import pallas as pl
from jax.experimental.pallas import tpu as pltpu
from jax.experimental.pallas import tpu_sc as plsc

_EPS = 1e-5
_B = 16384
_F = 26
_V = 100000
_D = 50
_NC = 13
_FD = _F * _D
_L1 = 512
_L2 = 256

_DP = 56
_FDP = _F * _DP

_ROWS = _B * _F
_NUM_WORKERS = 32
_ROWS_PER_W = _ROWS // _NUM_WORKERS
_CHUNK = 832
_NCHUNKS = _ROWS_PER_W // _CHUNK


def _sc_gather(flat_tables, flat_idx):
    mesh = plsc.VectorSubcoreMesh(core_axis_name="c", subcore_axis_name="s")

    @functools.partial(
        pl.kernel,
        out_type=jax.ShapeDtypeStruct((_ROWS, _DP), jnp.float32),
        mesh=mesh,
        scratch_types=[
            pltpu.VMEM((_ROWS_PER_W,), jnp.int32),
            pltpu.VMEM((_CHUNK, _DP), jnp.float32),
            pltpu.SemaphoreType.DMA,
        ],
        compiler_params=pltpu.CompilerParams(use_tc_tiling_on_sc=False),
    )
    def gather_kernel(tab_hbm, idx_hbm, out_hbm, idx_v, rows_v, sem):
        wid = lax.axis_index("s") * 2 + lax.axis_index("c")
        base = wid * _ROWS_PER_W
        pltpu.sync_copy(idx_hbm.at[pl.ds(base, _ROWS_PER_W)], idx_v)

        @pl.loop(0, _NCHUNKS)
        def _(i):
            off = i * _CHUNK
            pltpu.async_copy(
                tab_hbm.at[idx_v.at[pl.ds(off, _CHUNK)]], rows_v, sem
            ).wait()
            pltpu.sync_copy(rows_v, out_hbm.at[pl.ds(base + off, _CHUNK)])

    return gather_kernel(flat_tables, flat_idx)


def _mlp_body(x_ref, xc_ref, gc_ref, bc_ref, w1a_ref, w1b_ref, b1_ref,
              g1_ref, bt1_ref, w2_ref, b2_ref, g2_ref, bt2_ref, wo_ref,
              bo_ref, o_ref):
    inv = (1.0 / jnp.sqrt(1.0 + _EPS)).astype(jnp.float32)
    xc = xc_ref[...] * (gc_ref[...] * inv) + bc_ref[...]
    h = jnp.dot(x_ref[...], w1a_ref[...], preferred_element_type=jnp.float32)
    h = h + jnp.dot(xc, w1b_ref[...], preferred_element_type=jnp.float32)
    h = jnp.maximum(h + b1_ref[...], 0.0)
    h = h * (g1_ref[...] * inv) + bt1_ref[...]
    h = jnp.maximum(
        jnp.dot(h, w2_ref[...], preferred_element_type=jnp.float32)
        + b2_ref[...], 0.0)
    h = h * (g2_ref[...] * inv) + bt2_ref[...]
    o_ref[...] = (
        jnp.dot(h, wo_ref[...], preferred_element_type=jnp.float32)
        + bo_ref[...])


def _tc_mlp(x, x_cont, g_cont, b_cont, W1, b1, g1, beta1, W2, b2, g2, beta2,
            Wo, bo):
    bt = 1024
    grid = (_B // bt,)
    row = lambda v: v.reshape(1, -1)
    full = lambda a: pl.BlockSpec(a.shape, lambda i: (0, 0))
    w1a = jnp.pad(W1[:_FD].reshape(_F, _D, _L1),
                  ((0, 0), (0, _DP - _D), (0, 0))).reshape(_FDP, _L1)
    args = (
        x, x_cont, row(g_cont), row(b_cont),
        w1a, W1[_FD:], row(b1), row(g1), row(beta1),
        W2, row(b2), row(g2), row(beta2), Wo, row(bo),
    )
    in_specs = [
        pl.BlockSpec((bt, _FDP), lambda i: (i, 0)),
        pl.BlockSpec((bt, _NC), lambda i: (i, 0)),
    ] + [full(a) for a in args[2:]]
    return pl.pallas_call(
        _mlp_body,
        grid=grid,
        in_specs=in_specs,
        out_specs=pl.BlockSpec((bt, 1), lambda i: (i, 0)),
        out_shape=jax.ShapeDtypeStruct((_B, 1), jnp.float32),
        compiler_params=pltpu.CompilerParams(
            dimension_semantics=("arbitrary",)),
    )(*args)


def kernel(x_cat, x_cont, tables, g_cont, b_cont, W1, b1, g1, beta1, W2, b2,
           g2, beta2, Wo, bo):
    flat_tables = jnp.pad(tables.reshape(_F * _V, _D),
                          ((0, 0), (0, _DP - _D)))
    offs = (jnp.arange(_F, dtype=jnp.int32) * _V)[None, :]
    flat_idx = (x_cat.astype(jnp.int32) + offs).reshape(_ROWS)
    emb = _sc_gather(flat_tables, flat_idx)
    x = emb.reshape(_B, _FDP)
    return _tc_mlp(x, x_cont, g_cont, b_cont, W1, b1, g1, beta1, W2, b2, g2,
                   beta2, Wo, bo)


# speedup vs baseline: 12.7229x; 3.0086x over previous
import functools

import jax
import jax.numpy as jnp
from jax import lax
from jax.experimental import pallas as pl
from jax.experimental.pallas import tpu as pltpu
from jax.experimental.pallas import tpu_sc as plsc

_EPS = 1e-5
_B = 16384
_F = 26
_V = 100000
_D = 50
_NC = 13
_L1 = 512
_L2 = 256

_DP = 128
_FDP = _F * _DP
_FV = _F * _V

_ROWS = _B * _F
_NUM_WORKERS = 32
_ROWS_PER_W = _ROWS // _NUM_WORKERS
_CHUNK = 256
_NCHUNKS = _ROWS_PER_W // _CHUNK


def _sc_gather(tab128, flat_idx):
    mesh = plsc.VectorSubcoreMesh(core_axis_name="c", subcore_axis_name="s")

    @functools.partial(
        pl.kernel,
        out_type=jax.ShapeDtypeStruct((_ROWS, _DP), jnp.float32),
        mesh=mesh,
        scratch_types=[
            pltpu.VMEM((_ROWS_PER_W,), jnp.int32),
            pltpu.VMEM((_CHUNK, _DP), jnp.float32),
            pltpu.VMEM((_CHUNK, _DP), jnp.float32),
            pltpu.SemaphoreType.DMA,
            pltpu.SemaphoreType.DMA,
        ],
        compiler_params=pltpu.CompilerParams(use_tc_tiling_on_sc=True),
    )
    def gather_kernel(tab_hbm, idx_hbm, out_hbm, idx_v, buf0, buf1, sem0,
                      sem1):
        wid = lax.axis_index("s") * 2 + lax.axis_index("c")
        base = wid * _ROWS_PER_W
        pltpu.sync_copy(idx_hbm.at[pl.ds(base, _ROWS_PER_W)], idx_v)

        def start(i, buf, sem):
            pltpu.async_copy(
                tab_hbm.at[idx_v.at[pl.ds(i * _CHUNK, _CHUNK)]], buf, sem)

        def finish(i, buf, sem):
            pltpu.make_async_copy(
                tab_hbm.at[idx_v.at[pl.ds(i * _CHUNK, _CHUNK)]], buf, sem
            ).wait()
            pltpu.sync_copy(buf, out_hbm.at[pl.ds(base + i * _CHUNK, _CHUNK)])

        start(0, buf0, sem0)

        @pl.loop(0, _NCHUNKS, step=2)
        def _(i):
            start(i + 1, buf1, sem1)
            finish(i, buf0, sem0)

            @pl.when(i + 2 < _NCHUNKS)
            def _():
                start(i + 2, buf0, sem0)

            finish(i + 1, buf1, sem1)

    return gather_kernel(tab128, flat_idx)


def _mlp_body(x_ref, xc_ref, gc_ref, bc_ref, w1a_ref, w1b_ref, b1_ref,
              g1_ref, bt1_ref, w2_ref, b2_ref, g2_ref, bt2_ref, wo_ref,
              bo_ref, o_ref):
    inv = (1.0 / jnp.sqrt(1.0 + _EPS)).astype(jnp.float32)
    xc = xc_ref[...] * (gc_ref[...] * inv) + bc_ref[...]
    h = jnp.dot(x_ref[...], w1a_ref[...], preferred_element_type=jnp.float32)
    h = h + jnp.dot(xc, w1b_ref[...], preferred_element_type=jnp.float32)
    h = jnp.maximum(h + b1_ref[...], 0.0)
    h = h * (g1_ref[...] * inv) + bt1_ref[...]
    h = jnp.maximum(
        jnp.dot(h, w2_ref[...], preferred_element_type=jnp.float32)
        + b2_ref[...], 0.0)
    h = h * (g2_ref[...] * inv) + bt2_ref[...]
    o_ref[...] = (
        jnp.dot(h, wo_ref[...], preferred_element_type=jnp.float32)
        + bo_ref[...])


def _tc_mlp(x, x_cont, g_cont, b_cont, W1, b1, g1, beta1, W2, b2, g2, beta2,
            Wo, bo):
    bt = 1024
    grid = (_B // bt,)
    row = lambda v: v.reshape(1, -1)
    w1a = jnp.pad(W1[:_F * _D].reshape(_F, _D, _L1),
                  ((0, 0), (0, _DP - _D), (0, 0))).reshape(_FDP, _L1)
    args = (
        x, x_cont, row(g_cont), row(b_cont),
        w1a, W1[_F * _D:], row(b1), row(g1), row(beta1),
        W2, row(b2), row(g2), row(beta2), Wo, row(bo),
    )
    full = lambda a: pl.BlockSpec(a.shape, lambda i: (0,) * a.ndim)
    in_specs = [
        pl.BlockSpec((bt, _FDP), lambda i: (i, 0)),
        pl.BlockSpec((bt, _NC), lambda i: (i, 0)),
    ] + [full(a) for a in args[2:]]
    return pl.pallas_call(
        _mlp_body,
        grid=grid,
        in_specs=in_specs,
        out_specs=pl.BlockSpec((bt, 1), lambda i: (i, 0)),
        out_shape=jax.ShapeDtypeStruct((_B, 1), jnp.float32),
        compiler_params=pltpu.CompilerParams(
            dimension_semantics=("arbitrary",)),
    )(*args)


def kernel(x_cat, x_cont, tables, g_cont, b_cont, W1, b1, g1, beta1, W2, b2,
           g2, beta2, Wo, bo):
    view = jnp.transpose(tables, (0, 2, 1))
    eye = jnp.eye(_D, _DP, dtype=jnp.float32)
    eye = lax.optimization_barrier(eye)
    tab128 = jnp.einsum("fdv,dc->fvc", view, eye,
                        precision=lax.Precision.DEFAULT).reshape(_FV, _DP)
    offs = (jnp.arange(_F, dtype=jnp.int32) * _V)[None, :]
    flat_idx = (x_cat.astype(jnp.int32) + offs).reshape(_ROWS)
    emb = _sc_gather(tab128, flat_idx)
    x = emb.reshape(_B, _FDP)
    return _tc_mlp(x, x_cont, g_cont, b_cont, W1, b1, g1, beta1, W2, b2, g2,
                   beta2, Wo, bo)


# speedup vs baseline: 15.1649x; 1.1919x over previous
import functools

import jax
import jax.numpy as jnp
from jax import lax
from jax.experimental import pallas as pl
from jax.experimental.pallas import tpu as pltpu
from jax.experimental.pallas import tpu_sc as plsc

_EPS = 1e-5
_B = 16384
_F = 26
_V = 100000
_D = 50
_NC = 13
_L1 = 512
_L2 = 256

_DP = 128
_FDP = _F * _DP
_FV = _F * _V

_ROWS = _B * _F
_NUM_WORKERS = 32
_ROWS_PER_W = _ROWS // _NUM_WORKERS
_CHUNK = 256
_NCHUNKS = _ROWS_PER_W // _CHUNK


def _sc_gather(tab128, flat_idx):
    mesh = plsc.VectorSubcoreMesh(core_axis_name="c", subcore_axis_name="s")

    @functools.partial(
        pl.kernel,
        out_type=jax.ShapeDtypeStruct((_ROWS, _DP), jnp.float32),
        mesh=mesh,
        scratch_types=[
            pltpu.VMEM((_ROWS_PER_W,), jnp.int32),
            pltpu.VMEM((_CHUNK, _DP), jnp.float32),
            pltpu.VMEM((_CHUNK, _DP), jnp.float32),
            pltpu.SemaphoreType.DMA,
            pltpu.SemaphoreType.DMA,
        ],
        compiler_params=pltpu.CompilerParams(use_tc_tiling_on_sc=True),
    )
    def gather_kernel(tab_hbm, idx_hbm, out_hbm, idx_v, buf0, buf1, sem0,
                      sem1):
        wid = lax.axis_index("s") * 2 + lax.axis_index("c")
        base = wid * _ROWS_PER_W
        pltpu.sync_copy(idx_hbm.at[pl.ds(base, _ROWS_PER_W)], idx_v)

        def start(i, buf, sem):
            pltpu.async_copy(
                tab_hbm.at[idx_v.at[pl.ds(i * _CHUNK, _CHUNK)]], buf, sem)

        def finish(i, buf, sem):
            pltpu.make_async_copy(
                tab_hbm.at[idx_v.at[pl.ds(i * _CHUNK, _CHUNK)]], buf, sem
            ).wait()
            pltpu.sync_copy(buf, out_hbm.at[pl.ds(base + i * _CHUNK, _CHUNK)])

        start(0, buf0, sem0)

        @pl.loop(0, _NCHUNKS, step=2)
        def _(i):
            start(i + 1, buf1, sem1)
            finish(i, buf0, sem0)

            @pl.when(i + 2 < _NCHUNKS)
            def _():
                start(i + 2, buf0, sem0)

            finish(i + 1, buf1, sem1)

    return gather_kernel(tab128, flat_idx)


def _mlp_body(x_ref, xc_ref, gc_ref, bc_ref, w1a_ref, w1b_ref, b1_ref,
              g1_ref, bt1_ref, w2_ref, b2_ref, g2_ref, bt2_ref, wo_ref,
              bo_ref, o_ref):
    inv = (1.0 / jnp.sqrt(1.0 + _EPS)).astype(jnp.float32)
    xc = xc_ref[...] * (gc_ref[...] * inv) + bc_ref[...]
    h = jnp.dot(x_ref[0], w1a_ref[0], preferred_element_type=jnp.float32)
    for f in range(1, _F):
        h = h + jnp.dot(x_ref[f], w1a_ref[f],
                        preferred_element_type=jnp.float32)
    h = h + jnp.dot(xc, w1b_ref[...], preferred_element_type=jnp.float32)
    h = jnp.maximum(h + b1_ref[...], 0.0)
    h = h * (g1_ref[...] * inv) + bt1_ref[...]
    h = jnp.maximum(
        jnp.dot(h, w2_ref[...], preferred_element_type=jnp.float32)
        + b2_ref[...], 0.0)
    h = h * (g2_ref[...] * inv) + bt2_ref[...]
    o_ref[...] = (
        jnp.dot(h, wo_ref[...], preferred_element_type=jnp.float32)
        + bo_ref[...])


def _tc_mlp(x, x_cont, g_cont, b_cont, W1, b1, g1, beta1, W2, b2, g2, beta2,
            Wo, bo):
    bt = 1024
    grid = (_B // bt,)
    row = lambda v: v.reshape(1, -1)
    w1a = jnp.pad(W1[:_F * _D].reshape(_F, _D, _L1),
                  ((0, 0), (0, _DP - _D), (0, 0)))
    args = (
        x, x_cont, row(g_cont), row(b_cont),
        w1a, W1[_F * _D:], row(b1), row(g1), row(beta1),
        W2, row(b2), row(g2), row(beta2), Wo, row(bo),
    )
    full = lambda a: pl.BlockSpec(a.shape, lambda i: (0,) * a.ndim)
    in_specs = [
        pl.BlockSpec((_F, bt, _DP), lambda i: (0, i, 0)),
        pl.BlockSpec((bt, _NC), lambda i: (i, 0)),
    ] + [full(a) for a in args[2:]]
    return pl.pallas_call(
        _mlp_body,
        grid=grid,
        in_specs=in_specs,
        out_specs=pl.BlockSpec((bt, 1), lambda i: (i, 0)),
        out_shape=jax.ShapeDtypeStruct((_B, 1), jnp.float32),
        compiler_params=pltpu.CompilerParams(
            dimension_semantics=("arbitrary",)),
    )(*args)


def kernel(x_cat, x_cont, tables, g_cont, b_cont, W1, b1, g1, beta1, W2, b2,
           g2, beta2, Wo, bo):
    view = jnp.transpose(tables, (0, 2, 1))
    eye = jnp.eye(_D, _DP, dtype=jnp.float32)
    eye = lax.optimization_barrier(eye)
    tab128 = jnp.einsum("fdv,dc->fvc", view, eye,
                        precision=lax.Precision.DEFAULT).reshape(_FV, _DP)
    offs = (jnp.arange(_F, dtype=jnp.int32) * _V)[:, None]
    flat_idx = (x_cat.astype(jnp.int32).T + offs).reshape(_ROWS)
    emb = _sc_gather(tab128, flat_idx)
    x = emb.reshape(_F, _B, _DP)
    return _tc_mlp(x, x_cont, g_cont, b_cont, W1, b1, g1, beta1, W2, b2, g2,
                   beta2, Wo, bo)
